# Initial kernel scaffold; baseline (speedup 1.0000x reference)
import jax
import jax.numpy as jnp
from jax.experimental import pallas as pl


def kernel(x, edge_index, batch, W0, b0, W1, b1, Wout, bout):
    raise NotImplementedError("write your pallas kernel here")


# speedup vs baseline: 16.6157x; 16.6157x over previous
import functools

import jax
import jax.numpy as jnp
from jax import lax
from jax.experimental import pallas as pl
from jax.experimental.pallas import tpu as pltpu
from jax.experimental.pallas import tpu_sc as plsc

NODES = 10000
FEAT = 128
EDGES = 320000
GRAPHS = 16

CORES = 2
SUBCORES = 16
TILES = CORES * SUBCORES
EDGES_PER_TILE = EDGES // TILES
CHUNK = 80
NCHUNKS = EDGES_PER_TILE // CHUNK
ROWS_PER_SUB = NODES // SUBCORES
DEG_W = 16

RBLK = 2000
GRID = NODES // RBLK


def _sc_mesh():
    return plsc.VectorSubcoreMesh(core_axis_name="c", subcore_axis_name="s")


def _sc_degree(dst_r, ones_chunk, zeros_deg):
    @functools.partial(
        pl.kernel,
        mesh=_sc_mesh(),
        out_type=(
            jax.ShapeDtypeStruct((NODES, DEG_W), jnp.float32),
            jax.ShapeDtypeStruct((NODES, DEG_W), jnp.float32),
        ),
        scratch_types=[
            pltpu.VMEM((NCHUNKS, CHUNK), jnp.int32),
            pltpu.VMEM((CHUNK, DEG_W), jnp.float32),
            pltpu.VMEM_SHARED((NODES, DEG_W), jnp.float32),
        ],
    )
    def k(dst_hbm, ones_hbm, zeros_hbm, out_a, out_b, dst_v, ones_v, deg_sp):
        c = lax.axis_index("c")
        s = lax.axis_index("s")
        wid = c * SUBCORES + s
        pltpu.sync_copy(dst_hbm.at[wid], dst_v)
        pltpu.sync_copy(ones_hbm, ones_v)

        @pl.when(s == 0)
        def _():
            pltpu.sync_copy(zeros_hbm, deg_sp)

        plsc.subcore_barrier()

        def body(j, carry):
            pltpu.sync_copy(ones_v, deg_sp.at[dst_v.at[j]], add=True)
            return carry

        lax.fori_loop(0, NCHUNKS, body, 0)
        plsc.subcore_barrier()

        @pl.when((s == 0) & (c == 0))
        def _():
            pltpu.sync_copy(deg_sp, out_a)

        @pl.when((s == 0) & (c == 1))
        def _():
            pltpu.sync_copy(deg_sp, out_b)

    return k(dst_r, ones_chunk, zeros_deg)


def _sc_scatter(y, src_r, dst_r, zeros_acc):
    @functools.partial(
        pl.kernel,
        mesh=_sc_mesh(),
        out_type=(
            jax.ShapeDtypeStruct((NODES, FEAT), jnp.float32),
            jax.ShapeDtypeStruct((NODES, FEAT), jnp.float32),
        ),
        scratch_types=[
            pltpu.VMEM((NCHUNKS, CHUNK), jnp.int32),
            pltpu.VMEM((NCHUNKS, CHUNK), jnp.int32),
            pltpu.VMEM((CHUNK, FEAT), jnp.float32),
            pltpu.VMEM_SHARED((NODES, FEAT), jnp.float32),
            pltpu.SemaphoreType.DMA,
        ],
    )
    def k(y_hbm, src_hbm, dst_hbm, zeros_hbm, out_a, out_b,
          src_v, dst_v, rows_v, acc_sp, sem):
        c = lax.axis_index("c")
        s = lax.axis_index("s")
        wid = c * SUBCORES + s
        pltpu.sync_copy(src_hbm.at[wid], src_v)
        pltpu.sync_copy(dst_hbm.at[wid], dst_v)

        @pl.when(s == 0)
        def _():
            pltpu.sync_copy(zeros_hbm, acc_sp)

        plsc.subcore_barrier()

        def body(j, carry):
            pltpu.async_copy(y_hbm.at[src_v.at[j]], rows_v, sem).wait()
            pltpu.sync_copy(rows_v, acc_sp.at[dst_v.at[j]], add=True)
            return carry

        lax.fori_loop(0, NCHUNKS, body, 0)
        plsc.subcore_barrier()

        @pl.when((s == 0) & (c == 0))
        def _():
            pltpu.sync_copy(acc_sp, out_a)

        @pl.when((s == 0) & (c == 1))
        def _():
            pltpu.sync_copy(acc_sp, out_b)

    return k(y, src_r, dst_r, zeros_acc)


def _dinv_of(dega_ref, degb_ref):
    deg = dega_ref[:, 0] + degb_ref[:, 0] + 1.0
    return lax.rsqrt(deg)


def _tc_first(dega, degb, x, W0):
    def body(dega_ref, degb_ref, x_ref, w_ref, y_ref):
        dinv = _dinv_of(dega_ref, degb_ref)
        xw = jnp.dot(x_ref[...], w_ref[...], preferred_element_type=jnp.float32)
        y_ref[...] = dinv[:, None] * xw

    return pl.pallas_call(
        body,
        grid=(GRID,),
        in_specs=[
            pl.BlockSpec((RBLK, DEG_W), lambda i: (i, 0)),
            pl.BlockSpec((RBLK, DEG_W), lambda i: (i, 0)),
            pl.BlockSpec((RBLK, FEAT), lambda i: (i, 0)),
            pl.BlockSpec((FEAT, FEAT), lambda i: (0, 0)),
        ],
        out_specs=pl.BlockSpec((RBLK, FEAT), lambda i: (i, 0)),
        out_shape=jax.ShapeDtypeStruct((NODES, FEAT), jnp.float32),
    )(dega, degb, x, W0)


def _tc_layer(dega, degb, acca, accb, yprev, brow, W):
    def body(dega_ref, degb_ref, aa_ref, ab_ref, y_ref, b_ref, w_ref, o_ref):
        dinv = _dinv_of(dega_ref, degb_ref)
        pre = dinv[:, None] * (aa_ref[...] + ab_ref[...] + y_ref[...]) + b_ref[...]
        h = jnp.maximum(pre, 0.0)
        o_ref[...] = dinv[:, None] * jnp.dot(
            h, w_ref[...], preferred_element_type=jnp.float32)

    return pl.pallas_call(
        body,
        grid=(GRID,),
        in_specs=[
            pl.BlockSpec((RBLK, DEG_W), lambda i: (i, 0)),
            pl.BlockSpec((RBLK, DEG_W), lambda i: (i, 0)),
            pl.BlockSpec((RBLK, FEAT), lambda i: (i, 0)),
            pl.BlockSpec((RBLK, FEAT), lambda i: (i, 0)),
            pl.BlockSpec((RBLK, FEAT), lambda i: (i, 0)),
            pl.BlockSpec((1, FEAT), lambda i: (0, 0)),
            pl.BlockSpec((FEAT, FEAT), lambda i: (0, 0)),
        ],
        out_specs=pl.BlockSpec((RBLK, FEAT), lambda i: (i, 0)),
        out_shape=jax.ShapeDtypeStruct((NODES, FEAT), jnp.float32),
    )(dega, degb, acca, accb, yprev, brow, W)


def _tc_final(dega, degb, acca, accb, yprev, brow, batch2d):
    def body(dega_ref, degb_ref, aa_ref, ab_ref, y_ref, b_ref, batch_ref, o_ref):
        dinv = _dinv_of(dega_ref, degb_ref)
        node = dinv[:, None] * (aa_ref[...] + ab_ref[...] + y_ref[...]) + b_ref[...]
        gids = lax.broadcasted_iota(jnp.int32, (1, GRAPHS), 1)
        onehot = (batch_ref[...] == gids).astype(jnp.float32)
        part = lax.dot_general(onehot, node, (((0,), (0,)), ((), ())),
                               preferred_element_type=jnp.float32)

        @pl.when(pl.program_id(0) == 0)
        def _():
            o_ref[...] = jnp.zeros_like(o_ref)

        o_ref[...] += part

    return pl.pallas_call(
        body,
        grid=(GRID,),
        in_specs=[
            pl.BlockSpec((RBLK, DEG_W), lambda i: (i, 0)),
            pl.BlockSpec((RBLK, DEG_W), lambda i: (i, 0)),
            pl.BlockSpec((RBLK, FEAT), lambda i: (i, 0)),
            pl.BlockSpec((RBLK, FEAT), lambda i: (i, 0)),
            pl.BlockSpec((RBLK, FEAT), lambda i: (i, 0)),
            pl.BlockSpec((1, FEAT), lambda i: (0, 0)),
            pl.BlockSpec((RBLK, 1), lambda i: (i, 0)),
        ],
        out_specs=pl.BlockSpec((GRAPHS, FEAT), lambda i: (0, 0)),
        out_shape=jax.ShapeDtypeStruct((GRAPHS, FEAT), jnp.float32),
    )(dega, degb, acca, accb, yprev, brow, batch2d)


def kernel(x, edge_index, batch, W0, b0, W1, b1, Wout, bout):
    src_r = edge_index[1].reshape(TILES, NCHUNKS, CHUNK)
    dst_r = edge_index[0].reshape(TILES, NCHUNKS, CHUNK)
    ones_chunk = jnp.ones((CHUNK, DEG_W), jnp.float32)
    zeros_deg = jnp.zeros((NODES, DEG_W), jnp.float32)
    zeros_acc = jnp.zeros((NODES, FEAT), jnp.float32)
    batch2d = batch.reshape(NODES, 1)
    b0r = b0.reshape(1, FEAT)
    b1r = b1.reshape(1, FEAT)
    boutr = bout.reshape(1, FEAT)

    dega, degb = _sc_degree(dst_r, ones_chunk, zeros_deg)
    y0 = _tc_first(dega, degb, x, W0)
    a0, p0 = _sc_scatter(y0, src_r, dst_r, zeros_acc)
    y1 = _tc_layer(dega, degb, a0, p0, y0, b0r, W1)
    a1, p1 = _sc_scatter(y1, src_r, dst_r, zeros_acc)
    y2 = _tc_layer(dega, degb, a1, p1, y1, b1r, Wout)
    a2, p2 = _sc_scatter(y2, src_r, dst_r, zeros_acc)
    return _tc_final(dega, degb, a2, p2, y2, boutr, batch2d)
